# baseline (speedup 1.0000x reference)
import functools

import jax
import jax.numpy as jnp
from jax import lax
from jax.experimental import pallas as pl
from jax.experimental.pallas import tpu as pltpu
from jax.experimental.pallas import tpu_sc as plsc

_NC = 2
_NS = 16
_NW = _NC * _NS
_C = 80


def _pad_nodes(n):
    return (n + 127) // 128 * 128


@functools.lru_cache(maxsize=None)
def _deg_kernel(n_nodes: int, n_edges: int):
    epw = n_edges // _NW
    nch = epw // _C
    n_pad = _pad_nodes(n_nodes)
    rps = n_pad // _NS
    mesh = plsc.VectorSubcoreMesh(core_axis_name="c", subcore_axis_name="s",
                                  num_cores=_NC, num_subcores=_NS)

    @functools.partial(
        pl.kernel, mesh=mesh,
        out_type=jax.ShapeDtypeStruct((_NC, n_pad, 16), jnp.float32),
        scratch_types=[
            pltpu.VMEM((nch, _C), jnp.int32),
            pltpu.VMEM((_C, 16), jnp.float32),
            pltpu.VMEM_SHARED((n_pad, 16), jnp.float32),
        ],
        compiler_params=pltpu.CompilerParams(use_tc_tiling_on_sc=False),
    )
    def deg(ei_hbm, zero_hbm, out_hbm, dst_v, ones_v, acc_sh):
        cid = lax.axis_index("c")
        sid = lax.axis_index("s")
        wid = cid * _NS + sid

        def fill(r, carry):
            ones_v[r, 0:16] = jnp.ones((16,), jnp.float32)
            return carry

        lax.fori_loop(0, _C, fill, 0)
        pltpu.sync_copy(zero_hbm.at[pl.ds(sid * rps, rps)],
                        acc_sh.at[pl.ds(sid * rps, rps)])
        pltpu.sync_copy(ei_hbm.at[1, wid], dst_v)
        plsc.subcore_barrier()

        def body(j, carry):
            pltpu.sync_copy(ones_v, acc_sh.at[dst_v.at[j]], add=True)
            return carry

        lax.fori_loop(0, nch, body, 0)
        plsc.subcore_barrier()
        pltpu.sync_copy(acc_sh.at[pl.ds(sid * rps, rps)],
                        out_hbm.at[cid, pl.ds(sid * rps, rps)])

    return deg


_NB = 5


@functools.lru_cache(maxsize=None)
def _agg_kernel(n_nodes: int, n_edges: int, d: int):
    epw = n_edges // _NW
    nch = epw // _C
    ngr = nch // _NB
    gr_rows = _NB * _C
    n_pad = _pad_nodes(n_nodes)
    rps = n_pad // _NS
    mesh = plsc.VectorSubcoreMesh(core_axis_name="c", subcore_axis_name="s",
                                  num_cores=_NC, num_subcores=_NS)

    @functools.partial(
        pl.kernel, mesh=mesh,
        out_type=jax.ShapeDtypeStruct((_NC, n_pad, d), jnp.float32),
        scratch_types=[
            pltpu.VMEM((nch, _C), jnp.int32),
            pltpu.VMEM((nch, _C), jnp.int32),
            pltpu.VMEM((gr_rows, d), jnp.float32),
            pltpu.VMEM((gr_rows, d), jnp.float32),
            pltpu.VMEM_SHARED((n_pad, d), jnp.float32),
            pltpu.SemaphoreType.DMA,
            pltpu.SemaphoreType.DMA,
        ],
        compiler_params=pltpu.CompilerParams(use_tc_tiling_on_sc=False),
    )
    def agg(hs_hbm, ei_hbm, zero_hbm, out_hbm,
            src_v, dst_v, rows_a, rows_b, acc_sh, sem_a, sem_b):
        cid = lax.axis_index("c")
        sid = lax.axis_index("s")
        wid = cid * _NS + sid

        pltpu.sync_copy(zero_hbm.at[pl.ds(sid * rps, rps)],
                        acc_sh.at[pl.ds(sid * rps, rps)])
        pltpu.sync_copy(ei_hbm.at[0, wid], src_v)
        pltpu.sync_copy(ei_hbm.at[1, wid], dst_v)
        plsc.subcore_barrier()

        def fire(g, buf, sem):
            for b in range(_NB):
                pltpu.async_copy(
                    hs_hbm.at[src_v.at[g * _NB + b]],
                    buf.at[pl.ds(b * _C, _C)], sem)

        def drain(buf, sem):
            pltpu.make_async_copy(hs_hbm.at[pl.ds(0, gr_rows)], buf,
                                  sem).wait()

        def scat(g, buf):
            for b in range(_NB):
                pltpu.sync_copy(buf.at[pl.ds(b * _C, _C)],
                                acc_sh.at[dst_v.at[g * _NB + b]], add=True)

        fire(0, rows_a, sem_a)

        def body(i, carry):
            ga = 2 * i
            fire(ga + 1, rows_b, sem_b)
            drain(rows_a, sem_a)
            scat(ga, rows_a)
            fire(ga + 2, rows_a, sem_a)
            drain(rows_b, sem_b)
            scat(ga + 1, rows_b)
            return carry

        lax.fori_loop(0, (ngr - 1) // 2, body, 0)
        drain(rows_a, sem_a)
        scat(ngr - 1, rows_a)
        plsc.subcore_barrier()
        pltpu.sync_copy(acc_sh.at[pl.ds(sid * rps, rps)],
                        out_hbm.at[cid, pl.ds(sid * rps, rps)])

    return agg


def _dinv_of(deg_ref):
    deg = deg_ref[0, :, 0:1] + deg_ref[1, :, 0:1] + 1.0
    return lax.rsqrt(deg)


def _tc_matmul(x_ref, w1_ref, h_ref):
    h_ref[:, :] = jnp.dot(x_ref[:, :], w1_ref[:, :],
                          preferred_element_type=jnp.float32)


def _tc_scale(deg_ref, h_ref, hs_ref):
    hs_ref[:, :] = h_ref[:, :] * _dinv_of(deg_ref)


def _tc_mid(deg_ref, agg_ref, hs_ref, bias_ref, w2_ref, out_ref):
    dinv = _dinv_of(deg_ref)
    z = dinv * (agg_ref[0] + agg_ref[1] + hs_ref[:, :]) + bias_ref[:, :]
    z = jnp.maximum(z, 0.0)
    out_ref[:, :] = jnp.dot(z, w2_ref[:, :],
                            preferred_element_type=jnp.float32) * dinv


def _tc_last(deg_ref, agg_ref, hs_ref, bias_ref, wh_ref, bh_ref,
             logit_ref, value_ref):
    dinv = _dinv_of(deg_ref)
    z = dinv * (agg_ref[0] + agg_ref[1] + hs_ref[:, :]) + bias_ref[:, :]
    z = jnp.maximum(z, 0.0)
    hv = jnp.dot(z, wh_ref[:, :], preferred_element_type=jnp.float32)
    hv = hv + bh_ref[:, :]
    logit_ref[:, :] = hv[:, 0:1]
    value_ref[:, :] = hv[:, 1:2]


_R = 2000


def _row_spec(w):
    return pl.BlockSpec((_R, w), lambda i: (i, 0))


def _part_spec(w):
    return pl.BlockSpec((2, _R, w), lambda i: (0, i, 0))


def _full_spec(h, w):
    return pl.BlockSpec((h, w), lambda i: (0, 0))


def kernel(x, edge_index, W1, b1, W2, b2, actor_w, actor_b, critic_w, critic_b):
    n, d_in = x.shape
    d_hid = W1.shape[1]
    e = edge_index.shape[1]
    epw = e // _NW
    ei4 = edge_index.astype(jnp.int32).reshape(2, _NW, epw // _C, _C)

    n_pad = _pad_nodes(n)
    zero16 = jnp.zeros((n_pad, 16), jnp.float32)
    zerod = jnp.zeros((n_pad, d_hid), jnp.float32)
    deg3 = _deg_kernel(n, e)(ei4, zero16)

    grid = (n // _R,)
    h1 = pl.pallas_call(
        _tc_matmul,
        grid=grid,
        in_specs=[_row_spec(d_in), _full_spec(d_in, d_hid)],
        out_specs=_row_spec(d_hid),
        out_shape=jax.ShapeDtypeStruct((n, d_hid), jnp.float32),
    )(x, W1)
    hs1 = pl.pallas_call(
        _tc_scale,
        grid=grid,
        in_specs=[_part_spec(16), _row_spec(d_hid)],
        out_specs=_row_spec(d_hid),
        out_shape=jax.ShapeDtypeStruct((n, d_hid), jnp.float32),
    )(deg3, h1)

    agg_fn = _agg_kernel(n, e, d_hid)
    agg1 = agg_fn(hs1, ei4, zerod)
    hs2 = pl.pallas_call(
        _tc_mid,
        grid=grid,
        in_specs=[_part_spec(16), _part_spec(d_hid), _row_spec(d_hid),
                  _full_spec(1, d_hid), _full_spec(d_hid, d_hid)],
        out_specs=_row_spec(d_hid),
        out_shape=jax.ShapeDtypeStruct((n, d_hid), jnp.float32),
    )(deg3, agg1, hs1, b1.reshape(1, d_hid), W2)

    agg2 = agg_fn(hs2, ei4, zerod)
    wh2 = jnp.concatenate([actor_w, critic_w], axis=1)
    bh = jnp.concatenate([actor_b, critic_b]).reshape(1, 2)
    logits, value = pl.pallas_call(
        _tc_last,
        grid=grid,
        in_specs=[_part_spec(16), _part_spec(d_hid), _row_spec(d_hid),
                  _full_spec(1, d_hid), _full_spec(d_hid, 2),
                  _full_spec(1, 2)],
        out_specs=[pl.BlockSpec((_R, 1), lambda i: (i, 0)),
                   pl.BlockSpec((_R, 1), lambda i: (i, 0))],
        out_shape=[jax.ShapeDtypeStruct((n, 1), jnp.float32),
                   jax.ShapeDtypeStruct((n, 1), jnp.float32)],
    )(deg3, agg2, hs2, b2.reshape(1, d_hid), wh2, bh)

    return logits[:, 0], value


# speedup vs baseline: 1.0032x; 1.0032x over previous
import functools

import jax
import jax.numpy as jnp
from jax import lax
from jax.experimental import pallas as pl
from jax.experimental.pallas import tpu as pltpu
from jax.experimental.pallas import tpu_sc as plsc

_NC = 2
_NS = 16
_NW = _NC * _NS
_C = 80


def _pad_nodes(n):
    return (n + 127) // 128 * 128


@functools.lru_cache(maxsize=None)
def _deg_kernel(n_nodes: int, n_edges: int):
    epw = n_edges // _NW
    nch = epw // _C
    n_pad = _pad_nodes(n_nodes)
    rps = n_pad // _NS
    mesh = plsc.VectorSubcoreMesh(core_axis_name="c", subcore_axis_name="s",
                                  num_cores=_NC, num_subcores=_NS)

    @functools.partial(
        pl.kernel, mesh=mesh,
        out_type=jax.ShapeDtypeStruct((_NC, n_pad, 16), jnp.float32),
        scratch_types=[
            pltpu.VMEM((nch, _C), jnp.int32),
            pltpu.VMEM((_C, 16), jnp.float32),
            pltpu.VMEM_SHARED((n_pad, 16), jnp.float32),
        ],
        compiler_params=pltpu.CompilerParams(use_tc_tiling_on_sc=False),
    )
    def deg(ei_hbm, zero_hbm, out_hbm, dst_v, ones_v, acc_sh):
        cid = lax.axis_index("c")
        sid = lax.axis_index("s")
        wid = cid * _NS + sid

        def fill(r, carry):
            ones_v[r, 0:16] = jnp.ones((16,), jnp.float32)
            return carry

        lax.fori_loop(0, _C, fill, 0)
        pltpu.sync_copy(zero_hbm.at[pl.ds(sid * rps, rps)],
                        acc_sh.at[pl.ds(sid * rps, rps)])
        pltpu.sync_copy(ei_hbm.at[1, wid], dst_v)
        plsc.subcore_barrier()

        def body(j, carry):
            pltpu.sync_copy(ones_v, acc_sh.at[dst_v.at[j]], add=True)
            return carry

        lax.fori_loop(0, nch, body, 0)
        plsc.subcore_barrier()
        pltpu.sync_copy(acc_sh.at[pl.ds(sid * rps, rps)],
                        out_hbm.at[cid, pl.ds(sid * rps, rps)])

    return deg


_NB = 5


@functools.lru_cache(maxsize=None)
def _agg_kernel(n_nodes: int, n_edges: int, d: int):
    epw = n_edges // _NW
    nch = epw // _C
    ngr = nch // _NB
    gr_rows = _NB * _C
    n_pad = _pad_nodes(n_nodes)
    rps = n_pad // _NS
    mesh = plsc.VectorSubcoreMesh(core_axis_name="c", subcore_axis_name="s",
                                  num_cores=_NC, num_subcores=_NS)

    @functools.partial(
        pl.kernel, mesh=mesh,
        out_type=jax.ShapeDtypeStruct((_NC, n_pad, d), jnp.float32),
        scratch_types=[
            pltpu.VMEM((nch, _C), jnp.int32),
            pltpu.VMEM((nch, _C), jnp.int32),
            pltpu.VMEM((gr_rows, d), jnp.float32),
            pltpu.VMEM((gr_rows, d), jnp.float32),
            pltpu.VMEM_SHARED((n_pad, d), jnp.float32),
            pltpu.SemaphoreType.DMA,
            pltpu.SemaphoreType.DMA,
        ],
        compiler_params=pltpu.CompilerParams(use_tc_tiling_on_sc=False),
    )
    def agg(hs_hbm, ei_hbm, zero_hbm, out_hbm,
            src_v, dst_v, rows_a, rows_b, acc_sh, sem_a, sem_b):
        cid = lax.axis_index("c")
        sid = lax.axis_index("s")
        wid = cid * _NS + sid

        pltpu.sync_copy(zero_hbm.at[pl.ds(sid * rps, rps)],
                        acc_sh.at[pl.ds(sid * rps, rps)])
        pltpu.sync_copy(ei_hbm.at[0, wid], src_v)
        pltpu.sync_copy(ei_hbm.at[1, wid], dst_v)
        plsc.subcore_barrier()

        def fire(g, buf, sem):
            for b in range(_NB):
                pltpu.async_copy(
                    hs_hbm.at[src_v.at[g * _NB + b]],
                    buf.at[pl.ds(b * _C, _C)], sem)

        def drain(buf, sem):
            pltpu.make_async_copy(hs_hbm.at[pl.ds(0, gr_rows)], buf,
                                  sem).wait()

        def scat(g, buf):
            for b in range(_NB):
                pltpu.sync_copy(buf.at[pl.ds(b * _C, _C)],
                                acc_sh.at[dst_v.at[g * _NB + b]], add=True)

        fire(0, rows_a, sem_a)

        def body(i, carry):
            ga = 2 * i
            fire(ga + 1, rows_b, sem_b)
            drain(rows_a, sem_a)
            scat(ga, rows_a)
            fire(ga + 2, rows_a, sem_a)
            drain(rows_b, sem_b)
            scat(ga + 1, rows_b)
            return carry

        lax.fori_loop(0, (ngr - 1) // 2, body, 0)
        drain(rows_a, sem_a)
        scat(ngr - 1, rows_a)
        plsc.subcore_barrier()
        pltpu.sync_copy(acc_sh.at[pl.ds(sid * rps, rps)],
                        out_hbm.at[cid, pl.ds(sid * rps, rps)])

    return agg


def _dinv_of(deg_ref):
    deg = deg_ref[0, :, 0:1] + deg_ref[1, :, 0:1] + 1.0
    return lax.rsqrt(deg)


def _tc_first(deg_ref, x_ref, w1_ref, hs_ref):
    h = jnp.dot(x_ref[:, :], w1_ref[:, :], preferred_element_type=jnp.float32)
    hs_ref[:, :] = h * _dinv_of(deg_ref)


def _tc_mid(deg_ref, agg_ref, hs_ref, bias_ref, w2_ref, out_ref):
    dinv = _dinv_of(deg_ref)
    z = dinv * (agg_ref[0] + agg_ref[1] + hs_ref[:, :]) + bias_ref[:, :]
    z = jnp.maximum(z, 0.0)
    out_ref[:, :] = jnp.dot(z, w2_ref[:, :],
                            preferred_element_type=jnp.float32) * dinv


def _tc_last(deg_ref, agg_ref, hs_ref, bias_ref, wh_ref, bh_ref,
             logit_ref, value_ref):
    dinv = _dinv_of(deg_ref)
    z = dinv * (agg_ref[0] + agg_ref[1] + hs_ref[:, :]) + bias_ref[:, :]
    z = jnp.maximum(z, 0.0)
    hv = jnp.dot(z, wh_ref[:, :], preferred_element_type=jnp.float32)
    hv = hv + bh_ref[:, :]
    logit_ref[:, :] = hv[:, 0:1]
    value_ref[:, :] = hv[:, 1:2]


_R = 2000


def _row_spec(w):
    return pl.BlockSpec((_R, w), lambda i: (i, 0))


def _part_spec(w):
    return pl.BlockSpec((2, _R, w), lambda i: (0, i, 0))


def _full_spec(h, w):
    return pl.BlockSpec((h, w), lambda i: (0, 0))


def kernel(x, edge_index, W1, b1, W2, b2, actor_w, actor_b, critic_w, critic_b):
    n, d_in = x.shape
    d_hid = W1.shape[1]
    e = edge_index.shape[1]
    epw = e // _NW
    ei4 = edge_index.astype(jnp.int32).reshape(2, _NW, epw // _C, _C)

    n_pad = _pad_nodes(n)
    zero16 = jnp.zeros((n_pad, 16), jnp.float32)
    zerod = jnp.zeros((n_pad, d_hid), jnp.float32)
    deg3 = _deg_kernel(n, e)(ei4, zero16)

    grid = (n // _R,)
    hs1 = pl.pallas_call(
        _tc_first,
        grid=grid,
        in_specs=[_part_spec(16), _row_spec(d_in), _full_spec(d_in, d_hid)],
        out_specs=_row_spec(d_hid),
        out_shape=jax.ShapeDtypeStruct((n, d_hid), jnp.float32),
    )(deg3, x, W1)

    agg_fn = _agg_kernel(n, e, d_hid)
    agg1 = agg_fn(hs1, ei4, zerod)
    hs2 = pl.pallas_call(
        _tc_mid,
        grid=grid,
        in_specs=[_part_spec(16), _part_spec(d_hid), _row_spec(d_hid),
                  _full_spec(1, d_hid), _full_spec(d_hid, d_hid)],
        out_specs=_row_spec(d_hid),
        out_shape=jax.ShapeDtypeStruct((n, d_hid), jnp.float32),
    )(deg3, agg1, hs1, b1.reshape(1, d_hid), W2)

    agg2 = agg_fn(hs2, ei4, zerod)
    wh2 = jnp.concatenate([actor_w, critic_w], axis=1)
    bh = jnp.concatenate([actor_b, critic_b]).reshape(1, 2)
    logits, value = pl.pallas_call(
        _tc_last,
        grid=grid,
        in_specs=[_part_spec(16), _part_spec(d_hid), _row_spec(d_hid),
                  _full_spec(1, d_hid), _full_spec(d_hid, 2),
                  _full_spec(1, 2)],
        out_specs=[pl.BlockSpec((_R, 1), lambda i: (i, 0)),
                   pl.BlockSpec((_R, 1), lambda i: (i, 0))],
        out_shape=[jax.ShapeDtypeStruct((n, 1), jnp.float32),
                   jax.ShapeDtypeStruct((n, 1), jnp.float32)],
    )(deg3, agg2, hs2, b2.reshape(1, d_hid), wh2, bh)

    return logits[:, 0], value
